# baseline (speedup 1.0000x reference)
import functools

import jax
import jax.numpy as jnp
from jax import lax
from jax.experimental import pallas as pl
from jax.experimental.pallas import tpu as pltpu
from jax.experimental.pallas import tpu_sc as plsc

NPOINT = 512
KNN = 16
N = 8192
B = 4
RB = 128
BIGI = 2**30


def _fps_body(xyz2d_ref, xyzs_ref, sel_ref):
    xs = [[xyz2d_ref[b, c] for c in range(3)] for b in range(B)]
    iota_f = (lax.broadcasted_iota(jnp.int32, (8, 1024), 0) * 1024
              + lax.broadcasted_iota(jnp.int32, (8, 1024), 1)
              ).astype(jnp.float32)

    def step(i, carry):
        out = []
        for b in range(B):
            dist, far = carry[b]
            x0, x1, x2 = xs[b]
            c0 = xyzs_ref[b, 0, far]
            c1 = xyzs_ref[b, 0, N + far]
            c2 = xyzs_ref[b, 0, 2 * N + far]
            sel_ref[b, pl.ds(i, 1), 0:1] = jnp.reshape(c0, (1, 1))
            sel_ref[b, pl.ds(i, 1), 1:2] = jnp.reshape(c1, (1, 1))
            sel_ref[b, pl.ds(i, 1), 2:3] = jnp.reshape(c2, (1, 1))
            d = (x0 - c0) ** 2 + (x1 - c1) ** 2
            d = d + (x2 - c2) ** 2
            dist = jnp.minimum(dist, d)
            mv = jnp.max(dist, axis=1, keepdims=True)
            m = jnp.max(mv, axis=0, keepdims=True)
            kf = jnp.where(dist == m, iota_f, jnp.float32(1e9))
            fv = jnp.min(kf, axis=1, keepdims=True)
            far = jnp.min(fv).astype(jnp.int32)
            out.append((dist, far))
        return tuple(out)

    dist0 = jnp.full((8, 1024), 1e10, dtype=jnp.float32)
    init = tuple((dist0, jnp.int32(0)) for _ in range(B))
    lax.fori_loop(0, NPOINT, step, init)


def _fps(xyz2d, xyzs):
    return pl.pallas_call(
        _fps_body,
        grid=(1,),
        in_specs=[
            pl.BlockSpec((B, 3, 8, 1024), lambda i: (0, 0, 0, 0)),
            pl.BlockSpec((B, 1, 3 * N), lambda i: (0, 0, 0),
                         memory_space=pltpu.SMEM),
        ],
        out_specs=pl.BlockSpec((B, NPOINT, 8), lambda i: (0, 0, 0)),
        out_shape=jax.ShapeDtypeStruct((B, NPOINT, 8), jnp.float32),
    )(xyz2d, xyzs)


def _knn_body(xyzT_ref, selT_ref, knn_ref):
    b = pl.program_id(0)
    xc0 = xyzT_ref[0, 0]
    xc1 = xyzT_ref[0, 1]
    xc2 = xyzT_ref[0, 2]
    s0 = selT_ref[0, 0, :].reshape(1, RB)
    s1 = selT_ref[0, 1, :].reshape(1, RB)
    s2 = selT_ref[0, 2, :].reshape(1, RB)
    dmat = (xc0 - s0) ** 2 + (xc1 - s1) ** 2
    dmat = dmat + (xc2 - s2) ** 2
    row_iota_f = lax.broadcasted_iota(jnp.int32, (N, RB), 0
                                      ).astype(jnp.float32)
    k_iota = lax.broadcasted_iota(jnp.int32, (KNN, RB), 0)
    base = b * N

    def pick(k, carry):
        dmat, acc = carry
        m = jnp.min(dmat, axis=0, keepdims=True)
        idxf = jnp.min(jnp.where(dmat == m, row_iota_f, jnp.float32(1e9)),
                       axis=0, keepdims=True)
        acc = jnp.where(k_iota == k, idxf.astype(jnp.int32) + base, acc)
        dmat = jnp.where(row_iota_f == idxf, jnp.float32(jnp.inf), dmat)
        return dmat, acc

    acc0 = jnp.zeros((KNN, RB), dtype=jnp.int32)
    _, acc = lax.fori_loop(0, KNN, pick, (dmat, acc0))
    knn_ref[0, :, :] = acc


def _knn(xyzT, selT):
    return pl.pallas_call(
        _knn_body,
        grid=(B, NPOINT // RB),
        in_specs=[
            pl.BlockSpec((1, 3, N, 1), lambda b, r: (b, 0, 0, 0)),
            pl.BlockSpec((1, 8, RB), lambda b, r: (b, 0, r)),
        ],
        out_specs=pl.BlockSpec((1, KNN, RB), lambda b, r: (b, 0, r)),
        out_shape=jax.ShapeDtypeStruct((B, KNN, NPOINT), jnp.int32),
        compiler_params=pltpu.CompilerParams(
            dimension_semantics=("parallel", "parallel")),
    )(xyzT, selT)


_NC = 2
_NS = 16
_NW = _NC * _NS
_ROWS = B * NPOINT * KNN
_PER_W = _ROWS // _NW


@functools.lru_cache(maxsize=1)
def _make_sc_gather():
    @functools.partial(
        pl.kernel,
        mesh=plsc.VectorSubcoreMesh(core_axis_name="c", subcore_axis_name="s"),
        out_type=jax.ShapeDtypeStruct((_ROWS, 16), jnp.float32),
        scratch_types=[
            pltpu.VMEM((_PER_W,), jnp.int32),
            pltpu.VMEM((_PER_W, 16), jnp.float32),
            pltpu.SemaphoreType.DMA,
        ],
        compiler_params=pltpu.CompilerParams(use_tc_tiling_on_sc=False),
    )
    def _sc_gather(table_hbm, idx_hbm, out_hbm, idx_v, rows_v, sem):
        wid = lax.axis_index("s") * _NC + lax.axis_index("c")
        base = wid * _PER_W
        pltpu.sync_copy(idx_hbm.at[pl.ds(base, _PER_W)], idx_v)
        pltpu.async_copy(table_hbm.at[idx_v], rows_v, sem).wait()
        pltpu.sync_copy(rows_v, out_hbm.at[pl.ds(base, _PER_W)])

    return _sc_gather


def kernel(x):
    xyz = jnp.transpose(x[:, :, :3], (0, 2, 1))
    xyz2d = xyz.reshape(B, 3, 8, 1024)
    xyzT = xyz.reshape(B, 3, N, 1)
    xyzs = xyz.reshape(B, 1, 3 * N)
    sel = _fps(xyz2d, xyzs)
    selT = jnp.transpose(sel, (0, 2, 1))
    knn_t = _knn(xyzT, selT)
    knn_idx = jnp.transpose(knn_t, (0, 2, 1))
    table = x.reshape(B * N, 16)
    rows = _make_sc_gather()(table, knn_idx.reshape(_ROWS))
    return rows.reshape(B, NPOINT, KNN, 16)


# speedup vs baseline: 1.6634x; 1.6634x over previous
import functools

import jax
import jax.numpy as jnp
from jax import lax
from jax.experimental import pallas as pl
from jax.experimental.pallas import tpu as pltpu
from jax.experimental.pallas import tpu_sc as plsc

NPOINT = 512
KNN = 16
N = 8192
B = 4
RB = 128
BIGI = 2**30


def _fps_body(xyz2d_ref, xyzs_ref, sel_ref):
    xs = [[xyz2d_ref[b, c] for c in range(3)] for b in range(B)]
    iota_f = (lax.broadcasted_iota(jnp.int32, (8, 1024), 0) * 1024
              + lax.broadcasted_iota(jnp.int32, (8, 1024), 1)
              ).astype(jnp.float32)

    def step(i, carry):
        out = []
        for b in range(B):
            dist, far = carry[b]
            x0, x1, x2 = xs[b]
            c0 = xyzs_ref[b, 0, far]
            c1 = xyzs_ref[b, 0, N + far]
            c2 = xyzs_ref[b, 0, 2 * N + far]
            sel_ref[b, pl.ds(i, 1), 0:1] = jnp.reshape(c0, (1, 1))
            sel_ref[b, pl.ds(i, 1), 1:2] = jnp.reshape(c1, (1, 1))
            sel_ref[b, pl.ds(i, 1), 2:3] = jnp.reshape(c2, (1, 1))
            sel_ref[b, pl.ds(i, 1), 3:4] = jnp.reshape(
                far.astype(jnp.float32), (1, 1))
            d = (x0 - c0) ** 2 + (x1 - c1) ** 2
            d = d + (x2 - c2) ** 2
            dist = jnp.minimum(dist, d)
            mv = jnp.max(dist, axis=1, keepdims=True)
            m = jnp.max(mv, axis=0, keepdims=True)
            kf = jnp.where(dist == m, iota_f, jnp.float32(1e9))
            fv = jnp.min(kf, axis=1, keepdims=True)
            far = jnp.min(fv).astype(jnp.int32)
            out.append((dist, far))
        return tuple(out)

    dist0 = jnp.full((8, 1024), 1e10, dtype=jnp.float32)
    init = tuple((dist0, jnp.int32(0)) for _ in range(B))
    lax.fori_loop(0, NPOINT, step, init)


def _fps(xyz2d, xyzs):
    return pl.pallas_call(
        _fps_body,
        grid=(1,),
        in_specs=[
            pl.BlockSpec((B, 3, 8, 1024), lambda i: (0, 0, 0, 0)),
            pl.BlockSpec((B, 1, 3 * N), lambda i: (0, 0, 0),
                         memory_space=pltpu.SMEM),
        ],
        out_specs=pl.BlockSpec((B, NPOINT, 8), lambda i: (0, 0, 0)),
        out_shape=jax.ShapeDtypeStruct((B, NPOINT, 8), jnp.float32),
    )(xyz2d, xyzs)


def _knn_body(xyzf_ref, sel_ref, knn_ref):
    b = pl.program_id(0)
    xf0 = xyzf_ref[0, 0]
    xf1 = xyzf_ref[0, 1]
    xf2 = xyzf_ref[0, 2]
    col_iota_f = lax.broadcasted_iota(jnp.int32, (RB, N), 1
                                      ).astype(jnp.float32)
    k_iota = lax.broadcasted_iota(jnp.int32, (RB, KNN), 1)
    base = b * N

    s0 = sel_ref[0, :, 0:1]
    s1 = sel_ref[0, :, 1:2]
    s2 = sel_ref[0, :, 2:3]
    selff = sel_ref[0, :, 3:4]
    dmat = (s0 - xf0) ** 2 + (s1 - xf1) ** 2
    dmat = dmat + (s2 - xf2) ** 2

    def pick(k, carry):
        dmat, acc = carry
        m = jnp.min(dmat, axis=1, keepdims=True)
        idxf = jnp.min(jnp.where(dmat == m, col_iota_f, jnp.float32(1e9)),
                       axis=1, keepdims=True)
        acc = jnp.where(k_iota == k, idxf.astype(jnp.int32) + base, acc)
        dmat = jnp.where(col_iota_f == idxf, jnp.float32(jnp.inf), dmat)
        return dmat, acc

    acc0 = jnp.where(k_iota == 0, selff.astype(jnp.int32) + base, 0)
    dmat = jnp.where(col_iota_f == selff, jnp.float32(jnp.inf), dmat)
    _, acc = lax.fori_loop(1, KNN, pick, (dmat, acc0))
    knn_ref[0, :, :] = acc


def _knn(xyzf4, sel):
    return pl.pallas_call(
        _knn_body,
        grid=(B, NPOINT // RB),
        in_specs=[
            pl.BlockSpec((1, 3, 1, N), lambda b, r: (b, 0, 0, 0)),
            pl.BlockSpec((1, RB, 8), lambda b, r: (b, r, 0)),
        ],
        out_specs=pl.BlockSpec((1, RB, KNN), lambda b, r: (b, r, 0)),
        out_shape=jax.ShapeDtypeStruct((B, NPOINT, KNN), jnp.int32),
        compiler_params=pltpu.CompilerParams(
            dimension_semantics=("parallel", "parallel")),
    )(xyzf4, sel)


_NC = 2
_NS = 16
_NW = _NC * _NS
_ROWS = B * NPOINT * KNN
_PER_W = _ROWS // _NW


@functools.lru_cache(maxsize=1)
def _make_sc_gather():
    @functools.partial(
        pl.kernel,
        mesh=plsc.VectorSubcoreMesh(core_axis_name="c", subcore_axis_name="s"),
        out_type=jax.ShapeDtypeStruct((_ROWS, 16), jnp.float32),
        scratch_types=[
            pltpu.VMEM((_PER_W,), jnp.int32),
            pltpu.VMEM((_PER_W, 16), jnp.float32),
            pltpu.SemaphoreType.DMA,
        ],
        compiler_params=pltpu.CompilerParams(use_tc_tiling_on_sc=False),
    )
    def _sc_gather(table_hbm, idx_hbm, out_hbm, idx_v, rows_v, sem):
        wid = lax.axis_index("s") * _NC + lax.axis_index("c")
        base = wid * _PER_W
        pltpu.sync_copy(idx_hbm.at[pl.ds(base, _PER_W)], idx_v)
        pltpu.async_copy(table_hbm.at[idx_v], rows_v, sem).wait()
        pltpu.sync_copy(rows_v, out_hbm.at[pl.ds(base, _PER_W)])

    return _sc_gather


def kernel(x):
    xyz = jnp.transpose(x[:, :, :3], (0, 2, 1))
    xyz2d = xyz.reshape(B, 3, 8, 1024)
    xyzf4 = xyz.reshape(B, 3, 1, N)
    xyzs = xyz.reshape(B, 1, 3 * N)
    sel = _fps(xyz2d, xyzs)
    knn_idx = _knn(xyzf4, sel)
    table = x.reshape(B * N, 16)
    rows = _make_sc_gather()(table, knn_idx.reshape(_ROWS))
    return rows.reshape(B, NPOINT, KNN, 16)
